# baseline (speedup 1.0000x reference)
import functools

import numpy as np

import jax
import jax.numpy as jnp
from jax import lax
from jax.experimental import pallas as pl
from jax.experimental.pallas import tpu as pltpu
from jax.experimental.pallas import tpu_sc as plsc


_SZ = 20
_F = 384
_BS, _DH, _DW = 32, 32, 32


def _interp_weight_matrix(out_size: int, in_size: int) -> np.ndarray:
    o = np.arange(out_size, dtype=np.float64)
    s = (o + 0.5) * (float(in_size) / float(out_size)) - 0.5
    s = np.maximum(s, 0.0)
    s0 = np.floor(s)
    frac = (s - s0).astype(np.float32)
    i0 = np.clip(s0.astype(np.int64), 0, in_size - 1)
    i1 = np.clip(s0.astype(np.int64) + 1, 0, in_size - 1)
    w = np.zeros((out_size, in_size), dtype=np.float32)
    w[np.arange(out_size), i0] += 1.0 - frac
    w[np.arange(out_size), i1] += frac
    return w


def _table_body(wmT_ref, rowT_ref, colT_ref, out_ref):
    xiT = jnp.dot(colT_ref[...], wmT_ref[...],
                  preferred_element_type=jnp.float32)
    yiT = jnp.dot(rowT_ref[...], wmT_ref[...],
                  preferred_element_type=jnp.float32)
    first = jnp.broadcast_to(xiT[:, None, :], (_F, _DH, _DW))
    second = jnp.broadcast_to(yiT[:, :, None], (_F, _DH, _DW))
    out_ref[...] = jnp.concatenate([first, second], axis=0)


def _table_flat_body(wmT_ref, rowT_ref, colT_ref, out_ref):
    xiT = jnp.dot(colT_ref[...], wmT_ref[...],
                  preferred_element_type=jnp.float32)
    yiT = jnp.dot(rowT_ref[...], wmT_ref[...],
                  preferred_element_type=jnp.float32)
    first = jnp.broadcast_to(xiT[:, None, :], (_F, _DH, _DW))
    second = jnp.broadcast_to(yiT[:, :, None], (_F, _DH, _DW))
    out_ref[...] = jnp.concatenate([first, second], axis=0).reshape(
        2 * _F, _DH * _DW)


def _bcast_flat_body(table_ref, out_ref):
    out_ref[...] = jnp.broadcast_to(table_ref[...][None], out_ref.shape)


_NC, _NS = 2, 16
_TW = 2 * _F * _DH * _DW


def _sc_fanout_body(table_hbm, out_hbm, table_sh):
    cid = lax.axis_index("c")
    sid = lax.axis_index("s")

    @pl.when(sid == 0)
    def _load():
        pltpu.sync_copy(table_hbm, table_sh)

    plsc.subcore_barrier()
    b = cid * _NS + sid
    pltpu.sync_copy(table_sh, out_hbm.at[b])


def kernel(row_w, col_w, bs, dh, dw):
    del bs, dh, dw
    wmT = jnp.asarray(_interp_weight_matrix(_DW, _SZ).T)

    table = pl.pallas_call(
        _table_flat_body,
        out_shape=jax.ShapeDtypeStruct((2 * _F, _DH * _DW), jnp.float32),
    )(wmT, row_w.T, col_w.T)

    fanout = pl.kernel(
        _sc_fanout_body,
        out_type=jax.ShapeDtypeStruct((_BS, _TW), jnp.float32),
        mesh=plsc.VectorSubcoreMesh(core_axis_name="c", subcore_axis_name="s"),
        scratch_types=[pltpu.VMEM_SHARED((_TW,), jnp.float32)],
        compiler_params=pltpu.CompilerParams(use_tc_tiling_on_sc=True),
    )
    out = fanout(table.reshape(_TW))
    return out.reshape(_BS, 2 * _F, _DH, _DW)


# speedup vs baseline: 17.4403x; 17.4403x over previous
import functools

import numpy as np

import jax
import jax.numpy as jnp
from jax import lax
from jax.experimental import pallas as pl
from jax.experimental.pallas import tpu as pltpu
from jax.experimental.pallas import tpu_sc as plsc


_SZ = 20
_F = 384
_BS, _DH, _DW = 32, 32, 32


def _interp_weight_matrix(out_size: int, in_size: int) -> np.ndarray:
    o = np.arange(out_size, dtype=np.float64)
    s = (o + 0.5) * (float(in_size) / float(out_size)) - 0.5
    s = np.maximum(s, 0.0)
    s0 = np.floor(s)
    frac = (s - s0).astype(np.float32)
    i0 = np.clip(s0.astype(np.int64), 0, in_size - 1)
    i1 = np.clip(s0.astype(np.int64) + 1, 0, in_size - 1)
    w = np.zeros((out_size, in_size), dtype=np.float32)
    w[np.arange(out_size), i0] += 1.0 - frac
    w[np.arange(out_size), i1] += frac
    return w


def _table_body(wmT_ref, rowT_ref, colT_ref, out_ref):
    xiT = jnp.dot(colT_ref[...], wmT_ref[...],
                  preferred_element_type=jnp.float32)
    yiT = jnp.dot(rowT_ref[...], wmT_ref[...],
                  preferred_element_type=jnp.float32)
    first = jnp.broadcast_to(xiT[:, None, :], (_F, _DH, _DW))
    second = jnp.broadcast_to(yiT[:, :, None], (_F, _DH, _DW))
    out_ref[...] = jnp.concatenate([first, second], axis=0)


def _tableT_body(wm_ref, row_ref, col_ref, out_ref):
    xi = jnp.dot(wm_ref[...], col_ref[...],
                 preferred_element_type=jnp.float32)
    yi = jnp.dot(wm_ref[...], row_ref[...],
                 preferred_element_type=jnp.float32)
    first = jnp.broadcast_to(xi[None, :, :], (_DH, _DW, _F))
    second = jnp.broadcast_to(yi[:, None, :], (_DH, _DW, _F))
    out_ref[...] = jnp.concatenate([first, second], axis=2).reshape(
        _DH * _DW, 2 * _F)


def _bcast_flat_body(table_ref, out_ref):
    out_ref[...] = jnp.broadcast_to(table_ref[...][None], out_ref.shape)


_NC, _NS = 2, 16
_TW = 2 * _F * _DH * _DW


def _sc_fanout_body(table_hbm, out_hbm, table_sh):
    cid = lax.axis_index("c")
    sid = lax.axis_index("s")

    @pl.when(sid == 0)
    def _load():
        pltpu.sync_copy(table_hbm, table_sh)

    plsc.subcore_barrier()
    b = cid * _NS + sid
    pltpu.sync_copy(table_sh, out_hbm.at[b])


def kernel(row_w, col_w, bs, dh, dw):
    del bs, dh, dw
    wm = jnp.asarray(_interp_weight_matrix(_DW, _SZ))

    tableT = pl.pallas_call(
        _tableT_body,
        out_shape=jax.ShapeDtypeStruct((_DH * _DW, 2 * _F), jnp.float32),
    )(wm, row_w, col_w)

    _BB = 4
    out = pl.pallas_call(
        _bcast_flat_body,
        grid=(_BS // _BB,),
        in_specs=[pl.BlockSpec((_DH * _DW, 2 * _F), lambda b: (0, 0))],
        out_specs=pl.BlockSpec((_BB, _DH * _DW, 2 * _F), lambda b: (b, 0, 0)),
        out_shape=jax.ShapeDtypeStruct((_BS, _DH * _DW, 2 * _F), jnp.float32),
    )(tableT)
    return out.transpose(0, 2, 1).reshape(_BS, 2 * _F, _DH, _DW)
